# baseline (speedup 1.0000x reference)
import jax
import jax.numpy as jnp
from jax.experimental import pallas as pl
from jax.experimental.pallas import tpu as pltpu

_N = 16384
_F = 1024
_CH = 512
_STEPS = _N // _CH
_NBUF = 4


def _body(ptr_ref, state_ref, mem_hbm, out_hbm, ctx_ref, ibuf, obuf, acc_ref,
          in_sem, out_sem):
    ptr = ptr_ref[0]

    def in_dma(s, b):
        return pltpu.make_async_copy(
            mem_hbm.at[pl.ds(s * _CH, _CH)], ibuf.at[b], in_sem.at[b])

    def out_dma(s, b):
        return pltpu.make_async_copy(
            obuf.at[b], out_hbm.at[pl.ds(s * _CH, _CH)], out_sem.at[b])

    for b in range(_NBUF):
        in_dma(b, b).start()

    acc_ref[...] = jnp.zeros_like(acc_ref)

    def step(s, carry):
        b = jax.lax.rem(s, _NBUF)
        in_dma(s, b).wait()

        @pl.when(s >= _NBUF)
        def _drain():
            out_dma(s - _NBUF, b).wait()

        @pl.when(s == ptr // _CH)
        def _scatter():
            local = ptr % _CH
            ibuf[b, pl.ds(local, 1), :] = state_ref[...]

        obuf[b] = ibuf[b]
        acc_ref[...] += jnp.sum(ibuf[b], axis=0, keepdims=True)
        out_dma(s, b).start()

        nxt = s + _NBUF

        @pl.when(nxt < _STEPS)
        def _refill():
            in_dma(nxt, b).start()

        return carry

    jax.lax.fori_loop(0, _STEPS, step, 0, unroll=False)

    for b in range(_NBUF):
        s = _STEPS - _NBUF + b
        out_dma(s, jnp.int32(s % _NBUF)).wait()

    ctx_ref[...] = acc_ref[...] * (1.0 / _N)


def kernel(new_state, memory_buffer, memory_ptr):
    ptr = jnp.asarray(memory_ptr, jnp.int32).reshape((1,))
    grid_spec = pltpu.PrefetchScalarGridSpec(
        num_scalar_prefetch=1,
        grid=(1,),
        in_specs=[
            pl.BlockSpec((1, _F), lambda i, p: (0, 0)),
            pl.BlockSpec(memory_space=pl.MemorySpace.ANY),
        ],
        out_specs=[
            pl.BlockSpec(memory_space=pl.MemorySpace.ANY),
            pl.BlockSpec((1, _F), lambda i, p: (0, 0)),
        ],
        scratch_shapes=[
            pltpu.VMEM((_NBUF, _CH, _F), jnp.float32),
            pltpu.VMEM((_NBUF, _CH, _F), jnp.float32),
            pltpu.VMEM((1, _F), jnp.float32),
            pltpu.SemaphoreType.DMA((_NBUF,)),
            pltpu.SemaphoreType.DMA((_NBUF,)),
        ],
    )
    mem_out, ctx = pl.pallas_call(
        _body,
        grid_spec=grid_spec,
        out_shape=[
            jax.ShapeDtypeStruct((_N, _F), jnp.float32),
            jax.ShapeDtypeStruct((1, _F), jnp.float32),
        ],
    )(ptr, new_state, memory_buffer)
    new_ptr = (memory_ptr + 1) % _N
    return (ctx.reshape(_F), mem_out, new_ptr)
